# baseline (speedup 1.0000x reference)
import functools

import jax
import jax.numpy as jnp
from jax import lax
from jax.experimental import pallas as pl
from jax.experimental.pallas import tpu as pltpu
from jax.experimental.pallas import tpu_sc as plsc

L, B, H = 2048, 2, 1024
K1, K2 = 5, 5
EPS = 1e-12
NEG = -1e30
LT = 256
GRID = L // LT


def _bdot(x, y):
    return jnp.dot(x.astype(jnp.bfloat16), y.astype(jnp.bfloat16),
                   preferred_element_type=jnp.float32)


def _start_logits_body(seq_ref, ws_ref, bs_ref, out_ref):
    x = seq_ref[...]
    sl = _bdot(x.reshape(LT * B, H), ws_ref[...])
    out_ref[...] = sl.reshape(LT, B) + bs_ref[0, 0]


def _start_logits(seq, ws, bs):
    return pl.pallas_call(
        _start_logits_body,
        grid=(GRID,),
        in_specs=[
            pl.BlockSpec((LT, B, H), lambda i: (i, 0, 0)),
            pl.BlockSpec((H, 1), lambda i: (0, 0)),
            pl.BlockSpec((1, 1), lambda i: (0, 0)),
        ],
        out_specs=pl.BlockSpec((LT, B), lambda i: (i, 0)),
        out_shape=jax.ShapeDtypeStruct((L, B), jnp.float32),
    )(seq, ws, bs)


def _topk_rows(x, k):
    r = x.shape[0]
    iota = lax.broadcasted_iota(jnp.int32, (r, L), 1)
    cur = x
    vals, idxs = [], []
    for _ in range(k):
        v = jnp.max(cur, axis=-1, keepdims=True)
        i = jnp.min(jnp.where(cur == v, iota, L), axis=-1, keepdims=True)
        vals.append(v)
        idxs.append(i)
        cur = jnp.where(iota == i, -1e38, cur)
    return jnp.concatenate(vals, 1), jnp.concatenate(idxs, 1)


def _start_select_body(sl_ref, pm_ref, stlp_ref, sti_ref, sp_ref):
    sl = sl_ref[...]
    pm = pm_ref[...].astype(jnp.float32)
    slm = sl * (1.0 - pm) + NEG * pm
    m = jnp.max(slm, axis=-1, keepdims=True)
    ex = jnp.exp(slm - m)
    se = jnp.sum(ex, axis=-1, keepdims=True)
    sp_ref[...] = ex / se
    slp = slm - m - jnp.log(se)
    vals, idxs = _topk_rows(slp, K1)
    stlp_ref[...] = jnp.concatenate(
        [vals, jnp.zeros((B, 128 - K1), jnp.float32)], 1)
    sti_ref[...] = jnp.concatenate(
        [idxs, jnp.zeros((B, 128 - K1), jnp.int32)], 1)


def _start_select(slm, pm):
    return pl.pallas_call(
        _start_select_body,
        in_specs=[
            pl.BlockSpec((B, L), lambda: (0, 0)),
            pl.BlockSpec((B, L), lambda: (0, 0)),
        ],
        out_specs=[
            pl.BlockSpec((B, 128), lambda: (0, 0)),
            pl.BlockSpec((B, 128), lambda: (0, 0)),
            pl.BlockSpec((B, L), lambda: (0, 0)),
        ],
        out_shape=[
            jax.ShapeDtypeStruct((B, 128), jnp.float32),
            jax.ShapeDtypeStruct((B, 128), jnp.int32),
            jax.ShapeDtypeStruct((B, L), jnp.float32),
        ],
    )(slm, pm)


def _sc_gather(seq2d, flat_idx):
    mesh = plsc.VectorSubcoreMesh(core_axis_name="c", subcore_axis_name="s")

    @functools.partial(
        pl.kernel,
        mesh=mesh,
        out_type=jax.ShapeDtypeStruct((16, H), jnp.float32),
        scratch_types=[
            pltpu.VMEM((16,), jnp.int32),
            pltpu.VMEM((16, H), jnp.float32),
            pltpu.SemaphoreType.DMA,
        ],
    )
    def gat(seq_hbm, idx_hbm, out_hbm, idx_v, rows_v, sem):
        cid = lax.axis_index("c")
        sid = lax.axis_index("s")

        @pl.when((cid == 0) & (sid == 0))
        def _():
            pltpu.sync_copy(idx_hbm, idx_v)
            pltpu.async_copy(seq_hbm.at[idx_v], rows_v, sem).wait()
            pltpu.sync_copy(rows_v, out_hbm)

    return gat(seq2d, flat_idx)


def _cond_feat_body(rows_ref, we0b_ref, be0_ref, out_ref):
    out_ref[...] = _bdot(rows_ref[...], we0b_ref[...]) + be0_ref[...]


def _cond_feat(rows16, we0b, be0):
    return pl.pallas_call(
        _cond_feat_body,
        in_specs=[
            pl.BlockSpec((16, H), lambda: (0, 0)),
            pl.BlockSpec((H, H), lambda: (0, 0)),
            pl.BlockSpec((1, H), lambda: (0, 0)),
        ],
        out_specs=pl.BlockSpec((16, H), lambda: (0, 0)),
        out_shape=jax.ShapeDtypeStruct((16, H), jnp.float32),
    )(rows16, we0b, be0)


def _end_logits_body(seq_ref, we0a_ref, c_ref, gamma_ref, beta_ref, we1_ref,
                     be1_ref, sp_ref, el_ref, sf_ref):
    x = seq_ref[...]
    a3 = _bdot(x.reshape(LT * B, H), we0a_ref[...]).reshape(LT, B, H)
    g = gamma_ref[...]
    bt = beta_ref[...]
    w1 = we1_ref[...]
    be1 = be1_ref[0, 0]
    cols = []
    for b in range(B):
        ab = a3[:, b, :]
        for k in range(K1):
            cbk = c_ref[b * K1 + k, :].reshape(1, H)
            t = jnp.tanh(ab + cbk)
            mu = jnp.mean(t, -1, keepdims=True)
            var = jnp.mean((t - mu) * (t - mu), -1, keepdims=True)
            ln = (t - mu) * lax.rsqrt(var + EPS) * g + bt
            cols.append(_bdot(ln, w1) + be1)
    el_ref[...] = jnp.concatenate(cols, 1)

    sp = sp_ref[...]
    contrib = jnp.concatenate(
        [_bdot(sp[b:b + 1, :], x[:, b, :]) for b in range(B)], 0)

    @pl.when(pl.program_id(0) == 0)
    def _():
        sf_ref[...] = jnp.zeros_like(sf_ref)

    sf_ref[...] += contrib


def _end_logits(seq, we0a, c16, gamma, beta, we1, be1, sp):
    return pl.pallas_call(
        _end_logits_body,
        grid=(GRID,),
        in_specs=[
            pl.BlockSpec((LT, B, H), lambda i: (i, 0, 0)),
            pl.BlockSpec((H, H), lambda i: (0, 0)),
            pl.BlockSpec((16, H), lambda i: (0, 0)),
            pl.BlockSpec((1, H), lambda i: (0, 0)),
            pl.BlockSpec((1, H), lambda i: (0, 0)),
            pl.BlockSpec((H, 1), lambda i: (0, 0)),
            pl.BlockSpec((1, 1), lambda i: (0, 0)),
            pl.BlockSpec((B, LT), lambda i: (0, i)),
        ],
        out_specs=[
            pl.BlockSpec((LT, B * K1), lambda i: (i, 0)),
            pl.BlockSpec((B, H), lambda i: (0, 0)),
        ],
        out_shape=[
            jax.ShapeDtypeStruct((L, B * K1), jnp.float32),
            jax.ShapeDtypeStruct((B, H), jnp.float32),
        ],
        compiler_params=pltpu.CompilerParams(
            dimension_semantics=("arbitrary",)),
    )(seq, we0a, c16, gamma, beta, we1, be1, sp)


def _finish_body(el_ref, pm_ref, sf_ref, rows_ref, wa0_ref, ba0_ref, wa1_ref,
                 etlp_ref, eti_ref, cls_ref):
    el = el_ref[...]
    pm = pm_ref[...].astype(jnp.float32)
    pm10 = jnp.concatenate(
        [jnp.broadcast_to(pm[b:b + 1, :], (K1, L)) for b in range(B)], 0)
    elm = el * (1.0 - pm10) + NEG * pm10
    m = jnp.max(elm, -1, keepdims=True)
    se = jnp.sum(jnp.exp(elm - m), -1, keepdims=True)
    elp = elm - m - jnp.log(se)
    vals, idxs = _topk_rows(elp, K2)
    etlp_ref[...] = jnp.concatenate(
        [vals, jnp.zeros((B * K1, 128 - K2), jnp.float32)], 1)
    eti_ref[...] = jnp.concatenate(
        [idxs, jnp.zeros((B * K1, 128 - K2), jnp.int32)], 1)

    af = jnp.concatenate([sf_ref[...], rows_ref[2 * K1:2 * K1 + B, :]], 1)
    h1 = jnp.tanh(_bdot(af, wa0_ref[...]) + ba0_ref[...])
    cl = _bdot(h1, wa1_ref[...])
    cls_ref[...] = jnp.concatenate([cl, jnp.zeros((B, 127), jnp.float32)], 1)


def _finish(el10, pm, sf, rows16, wa0, ba0, wa1):
    return pl.pallas_call(
        _finish_body,
        in_specs=[
            pl.BlockSpec((B * K1, L), lambda: (0, 0)),
            pl.BlockSpec((B, L), lambda: (0, 0)),
            pl.BlockSpec((B, H), lambda: (0, 0)),
            pl.BlockSpec((16, H), lambda: (0, 0)),
            pl.BlockSpec((2 * H, H), lambda: (0, 0)),
            pl.BlockSpec((1, H), lambda: (0, 0)),
            pl.BlockSpec((H, 1), lambda: (0, 0)),
        ],
        out_specs=[
            pl.BlockSpec((B * K1, 128), lambda: (0, 0)),
            pl.BlockSpec((B * K1, 128), lambda: (0, 0)),
            pl.BlockSpec((B, 128), lambda: (0, 0)),
        ],
        out_shape=[
            jax.ShapeDtypeStruct((B * K1, 128), jnp.float32),
            jax.ShapeDtypeStruct((B * K1, 128), jnp.int32),
            jax.ShapeDtypeStruct((B, 128), jnp.float32),
        ],
    )(el10, pm, sf, rows16, wa0, ba0, wa1)


def kernel(sequence_output, p_mask, Ws, bs, We0, be0, gamma, beta, We1, be1,
           Wa0, ba0, Wa1, start_n_top, end_n_top):
    seq = sequence_output.astype(jnp.float32)
    sl = _start_logits(seq, Ws, bs.reshape(1, 1))
    stlp_p, sti_p, start_p = _start_select(sl.T, p_mask)
    stlp = stlp_p[:, :K1]
    sti = sti_p[:, :K1]

    seq2d = seq.reshape(L * B, H)
    flat = (sti * B + jnp.arange(B, dtype=jnp.int32)[:, None]).reshape(-1)
    flat = jnp.concatenate(
        [flat, jnp.arange(B, dtype=jnp.int32),
         jnp.zeros((16 - B * K1 - B,), jnp.int32)])
    rows16 = _sc_gather(seq2d, flat)

    c16 = _cond_feat(rows16, We0[H:], be0.reshape(1, H))
    el_cols, sf = _end_logits(seq, We0[:H], c16, gamma.reshape(1, H),
                              beta.reshape(1, H), We1, be1.reshape(1, 1),
                              start_p)
    etlp_p, eti_p, cls_p = _finish(el_cols.T, p_mask, sf, rows16, Wa0,
                                   ba0.reshape(1, H), Wa1)
    etlp = etlp_p[:, :K2].reshape(B, K1 * K2)
    eti = eti_p[:, :K2].reshape(B, K1 * K2)
    cls_logits = cls_p[:, 0]
    return (stlp, sti, etlp, eti, cls_logits)


# speedup vs baseline: 2.2456x; 2.2456x over previous
import functools

import jax
import jax.numpy as jnp
from jax import lax
from jax.experimental import pallas as pl
from jax.experimental.pallas import tpu as pltpu
from jax.experimental.pallas import tpu_sc as plsc

L, B, H = 2048, 2, 1024
K1, K2 = 5, 5
BK = B * K1
EPS = 1e-12
NEG = -1e30
LT = 512
GRID = L // LT


def _start_logits_body(seq_ref, ws_ref, bs_ref, out_ref):
    x = seq_ref[...]
    sl = jnp.dot(x.reshape(LT * B, H), ws_ref[...],
                 preferred_element_type=jnp.float32)
    out_ref[...] = sl.reshape(LT, B) + bs_ref[0, 0]


def _start_logits(seq, ws, bs):
    return pl.pallas_call(
        _start_logits_body,
        grid=(GRID,),
        in_specs=[
            pl.BlockSpec((LT, B, H), lambda i: (i, 0, 0)),
            pl.BlockSpec((H, 1), lambda i: (0, 0)),
            pl.BlockSpec((1, 1), lambda i: (0, 0)),
        ],
        out_specs=pl.BlockSpec((LT, B), lambda i: (i, 0)),
        out_shape=jax.ShapeDtypeStruct((L, B), jnp.float32),
    )(seq, ws, bs)


def _topk_rows(x, k):
    r = x.shape[0]
    iota = lax.broadcasted_iota(jnp.int32, (r, L), 1)
    cur = x
    vals, idxs = [], []
    for _ in range(k):
        v = jnp.max(cur, axis=-1, keepdims=True)
        i = jnp.min(jnp.where(cur == v, iota, L), axis=-1, keepdims=True)
        vals.append(v)
        idxs.append(i)
        cur = jnp.where(iota == i, -1e38, cur)
    return jnp.concatenate(vals, 1), jnp.concatenate(idxs, 1)


def _start_select_body(sl_ref, pm_ref, stlp_ref, sti_ref, sp_ref, flat_ref):
    sl = sl_ref[...]
    pm = pm_ref[...].astype(jnp.float32)
    slm = sl * (1.0 - pm) + NEG * pm
    m = jnp.max(slm, axis=-1, keepdims=True)
    ex = jnp.exp(slm - m)
    se = jnp.sum(ex, axis=-1, keepdims=True)
    sp_ref[...] = ex / se
    slp = slm - m - jnp.log(se)
    vals, idxs = _topk_rows(slp, K1)
    stlp_ref[...] = jnp.concatenate(
        [vals, jnp.zeros((B, 128 - K1), jnp.float32)], 1)
    sti_ref[...] = jnp.concatenate(
        [idxs, jnp.zeros((B, 128 - K1), jnp.int32)], 1)
    fr = [idxs[b:b + 1, :] * B + b for b in range(B)]
    cls_r = lax.broadcasted_iota(jnp.int32, (1, B), 1)
    flat_ref[...] = jnp.concatenate(
        fr + [cls_r, jnp.zeros((1, 128 - B * K1 - B), jnp.int32)], 1)


def _start_select(slm, pm):
    return pl.pallas_call(
        _start_select_body,
        in_specs=[
            pl.BlockSpec((B, L), lambda: (0, 0)),
            pl.BlockSpec((B, L), lambda: (0, 0)),
        ],
        out_specs=[
            pl.BlockSpec((B, 128), lambda: (0, 0)),
            pl.BlockSpec((B, 128), lambda: (0, 0)),
            pl.BlockSpec((B, L), lambda: (0, 0)),
            pl.BlockSpec((1, 128), lambda: (0, 0)),
        ],
        out_shape=[
            jax.ShapeDtypeStruct((B, 128), jnp.float32),
            jax.ShapeDtypeStruct((B, 128), jnp.int32),
            jax.ShapeDtypeStruct((B, L), jnp.float32),
            jax.ShapeDtypeStruct((1, 128), jnp.int32),
        ],
    )(slm, pm)


def _sc_gather(seq2d, flat_idx):
    mesh = plsc.VectorSubcoreMesh(core_axis_name="c", subcore_axis_name="s")

    @functools.partial(
        pl.kernel,
        mesh=mesh,
        out_type=jax.ShapeDtypeStruct((16, H), jnp.float32),
        scratch_types=[
            pltpu.VMEM((16,), jnp.int32),
            pltpu.VMEM((16, H), jnp.float32),
            pltpu.SemaphoreType.DMA,
        ],
    )
    def gat(seq_hbm, idx_hbm, out_hbm, idx_v, rows_v, sem):
        cid = lax.axis_index("c")
        sid = lax.axis_index("s")

        @pl.when((cid == 0) & (sid == 0))
        def _():
            pltpu.sync_copy(idx_hbm.at[pl.ds(0, 16)], idx_v)
            pltpu.async_copy(seq_hbm.at[idx_v], rows_v, sem).wait()
            pltpu.sync_copy(rows_v, out_hbm)

    return gat(seq2d, flat_idx)


def _cond_feat_body(rows_ref, we0b_ref, be0_ref, out_ref):
    out_ref[...] = jnp.dot(
        rows_ref[...].astype(jnp.bfloat16), we0b_ref[...],
        preferred_element_type=jnp.float32) + be0_ref[...]


def _cond_feat(rows16, we0b, be0):
    return pl.pallas_call(
        _cond_feat_body,
        in_specs=[
            pl.BlockSpec((16, H), lambda: (0, 0)),
            pl.BlockSpec((H, H), lambda: (0, 0)),
            pl.BlockSpec((1, H), lambda: (0, 0)),
        ],
        out_specs=pl.BlockSpec((16, H), lambda: (0, 0)),
        out_shape=jax.ShapeDtypeStruct((16, H), jnp.float32),
    )(rows16, we0b, be0)


def _end_logits_body(seq_ref, we0a_ref, c_ref, gamma_ref, beta_ref, we1_ref,
                     be1_ref, sp_ref, el_ref, sf_ref):
    x2 = seq_ref[...].reshape(LT, H)
    a = jnp.dot(x2, we0a_ref[...],
                preferred_element_type=jnp.float32)
    g = gamma_ref[...]
    bt = beta_ref[...]
    w1 = we1_ref[...]
    be1 = be1_ref[0, 0]
    for k in range(K1):
        ck = c_ref[0, k, :].reshape(1, H)
        t = jnp.tanh(a + ck)
        mu = jnp.mean(t, -1, keepdims=True)
        u = t - mu
        var = jnp.mean(u * u, -1, keepdims=True)
        ln = u * lax.rsqrt(var + EPS) * g + bt
        col = jnp.dot(ln.astype(jnp.bfloat16), w1,
                      preferred_element_type=jnp.float32) + be1
        el_ref[:, 0, 0, k:k + 1] = col

    contrib = jnp.dot(sp_ref[...].reshape(1, LT).astype(jnp.bfloat16), x2,
                      preferred_element_type=jnp.float32)

    @pl.when(pl.program_id(1) == 0)
    def _():
        sf_ref[...] = jnp.zeros_like(sf_ref)

    sf_ref[...] += contrib.reshape(1, 1, H)


def _end_logits(seq4, we0a, c3, gamma, beta, we1, be1, sp3):
    return pl.pallas_call(
        _end_logits_body,
        grid=(B, GRID),
        in_specs=[
            pl.BlockSpec((LT, 1, 1, H), lambda b, l: (l, b, 0, 0)),
            pl.BlockSpec((H, H), lambda b, l: (0, 0)),
            pl.BlockSpec((1, K1, H), lambda b, l: (b, 0, 0)),
            pl.BlockSpec((1, H), lambda b, l: (0, 0)),
            pl.BlockSpec((1, H), lambda b, l: (0, 0)),
            pl.BlockSpec((H, 1), lambda b, l: (0, 0)),
            pl.BlockSpec((1, 1), lambda b, l: (0, 0)),
            pl.BlockSpec((1, 1, LT), lambda b, l: (b, 0, l)),
        ],
        out_specs=[
            pl.BlockSpec((LT, 1, 1, K1), lambda b, l: (l, b, 0, 0)),
            pl.BlockSpec((1, 1, H), lambda b, l: (b, 0, 0)),
        ],
        out_shape=[
            jax.ShapeDtypeStruct((L, B, 1, K1), jnp.float32),
            jax.ShapeDtypeStruct((B, 1, H), jnp.float32),
        ],
        compiler_params=pltpu.CompilerParams(
            dimension_semantics=("arbitrary", "arbitrary")),
    )(seq4, we0a, c3, gamma, beta, we1, be1, sp3)


def _finish_body(el_ref, pm_ref, sf_ref, rows_ref, wa0_ref, ba0_ref, wa1_ref,
                 etlp_ref, eti_ref, cls_ref):
    el = el_ref[...]
    pm = pm_ref[...].astype(jnp.float32)
    elm = el * (1.0 - pm) + NEG * pm
    m = jnp.max(elm, axis=0, keepdims=True)
    se = jnp.sum(jnp.exp(elm - m), axis=0, keepdims=True)
    elp = elm - m - jnp.log(se)
    iota = lax.broadcasted_iota(jnp.int32, (L, BK), 0)
    cur = elp
    vals, idxs = [], []
    for _ in range(K2):
        v = jnp.max(cur, axis=0, keepdims=True)
        i = jnp.min(jnp.where(cur == v, iota, L), axis=0, keepdims=True)
        vals.append(v)
        idxs.append(i)
        cur = jnp.where(iota == i, -1e38, cur)
    valc = jnp.concatenate(vals, 0)
    idxc = jnp.concatenate(idxs, 0)
    etlp_ref[...] = jnp.concatenate(
        [jnp.concatenate([valc, jnp.zeros((8 - K2, BK), jnp.float32)], 0),
         jnp.zeros((8, 128 - BK), jnp.float32)], 1)
    eti_ref[...] = jnp.concatenate(
        [jnp.concatenate([idxc, jnp.zeros((8 - K2, BK), jnp.int32)], 0),
         jnp.zeros((8, 128 - BK), jnp.int32)], 1)

    af = jnp.concatenate([sf_ref[...], rows_ref[BK:BK + B, :]], 1)
    h1 = jnp.tanh(jnp.dot(af.astype(jnp.bfloat16), wa0_ref[...],
                          preferred_element_type=jnp.float32) + ba0_ref[...])
    cl = jnp.dot(h1.astype(jnp.bfloat16), wa1_ref[...],
                 preferred_element_type=jnp.float32)
    cls_ref[...] = jnp.concatenate([cl, jnp.zeros((B, 127), jnp.float32)], 1)


def _finish(el10, pm10, sf, rows16, wa0, ba0, wa1):
    return pl.pallas_call(
        _finish_body,
        in_specs=[
            pl.BlockSpec((L, BK), lambda: (0, 0)),
            pl.BlockSpec((L, BK), lambda: (0, 0)),
            pl.BlockSpec((B, H), lambda: (0, 0)),
            pl.BlockSpec((16, H), lambda: (0, 0)),
            pl.BlockSpec((2 * H, H), lambda: (0, 0)),
            pl.BlockSpec((1, H), lambda: (0, 0)),
            pl.BlockSpec((H, 1), lambda: (0, 0)),
        ],
        out_specs=[
            pl.BlockSpec((8, 128), lambda: (0, 0)),
            pl.BlockSpec((8, 128), lambda: (0, 0)),
            pl.BlockSpec((B, 128), lambda: (0, 0)),
        ],
        out_shape=[
            jax.ShapeDtypeStruct((8, 128), jnp.float32),
            jax.ShapeDtypeStruct((8, 128), jnp.int32),
            jax.ShapeDtypeStruct((B, 128), jnp.float32),
        ],
    )(el10, pm10, sf, rows16, wa0, ba0, wa1)


def kernel(sequence_output, p_mask, Ws, bs, We0, be0, gamma, beta, We1, be1,
           Wa0, ba0, Wa1, start_n_top, end_n_top):
    seq = sequence_output.astype(jnp.float32)
    bf = jnp.bfloat16
    seq_bf = seq.astype(bf)
    sl = _start_logits(seq_bf, Ws.astype(bf), bs.reshape(1, 1))
    stlp_p, sti_p, start_p, flatp = _start_select(sl.T, p_mask)
    stlp = stlp_p[:, :K1]
    sti = sti_p[:, :K1]

    rows16 = _sc_gather(seq.reshape(L * B, H), flatp.reshape(128))

    c16 = _cond_feat(rows16, We0[H:].astype(bf), be0.reshape(1, H))
    c3 = c16[:BK].reshape(B, K1, H)
    el4, sf3 = _end_logits(seq_bf.reshape(L, B, 1, H), We0[:H].astype(bf), c3,
                           gamma.reshape(1, H), beta.reshape(1, H),
                           We1.astype(bf), be1.reshape(1, 1),
                           start_p.reshape(B, 1, L))
    el10 = el4.reshape(L, BK)
    pm10 = jnp.repeat(p_mask.T, K1, axis=1)
    etlp_p, eti_p, cls_p = _finish(el10, pm10, sf3.reshape(B, H), rows16,
                                   Wa0.astype(bf), ba0.reshape(1, H),
                                   Wa1.astype(bf))
    etlp = etlp_p[:K2, :BK].T.reshape(B, K1 * K2)
    eti = eti_p[:K2, :BK].T.reshape(B, K1 * K2)
    cls_logits = cls_p[:, 0]
    return (stlp, sti, etlp, eti, cls_logits)


# speedup vs baseline: 2.5190x; 1.1217x over previous
import functools

import jax
import jax.numpy as jnp
from jax import lax
from jax.experimental import pallas as pl
from jax.experimental.pallas import tpu as pltpu
from jax.experimental.pallas import tpu_sc as plsc

L, B, H = 2048, 2, 1024
K1, K2 = 5, 5
BK = B * K1
EPS = 1e-12
NEG = -1e30
LT = 1024
GRID = L // LT


def _start_logits_body(seq_ref, ws_ref, bs_ref, out_ref):
    x = seq_ref[...]
    sl = jnp.dot(x.reshape(LT * B, H), ws_ref[...],
                 preferred_element_type=jnp.float32)
    out_ref[...] = sl.reshape(LT, B) + bs_ref[0, 0]


def _start_logits(seq, ws, bs):
    return pl.pallas_call(
        _start_logits_body,
        grid=(GRID,),
        in_specs=[
            pl.BlockSpec((LT, B, H), lambda i: (i, 0, 0)),
            pl.BlockSpec((H, 1), lambda i: (0, 0)),
            pl.BlockSpec((1, 1), lambda i: (0, 0)),
        ],
        out_specs=pl.BlockSpec((LT, B), lambda i: (i, 0)),
        out_shape=jax.ShapeDtypeStruct((L, B), jnp.float32),
    )(seq, ws, bs)


def _topk_rows(x, k):
    r = x.shape[0]
    iota = lax.broadcasted_iota(jnp.int32, (r, L), 1)
    cur = x
    vals, idxs = [], []
    for _ in range(k):
        v = jnp.max(cur, axis=-1, keepdims=True)
        i = jnp.min(jnp.where(cur == v, iota, L), axis=-1, keepdims=True)
        vals.append(v)
        idxs.append(i)
        cur = jnp.where(iota == i, -1e38, cur)
    return jnp.concatenate(vals, 1), jnp.concatenate(idxs, 1)


def _start_select_body(sl_ref, pm_ref, stlp_ref, sti_ref, sp_ref, flat_ref):
    sl = sl_ref[...]
    pm = pm_ref[...].astype(jnp.float32)
    slm = sl * (1.0 - pm) + NEG * pm
    m = jnp.max(slm, axis=-1, keepdims=True)
    ex = jnp.exp(slm - m)
    se = jnp.sum(ex, axis=-1, keepdims=True)
    sp_ref[...] = ex / se
    slp = slm - m - jnp.log(se)
    vals, idxs = _topk_rows(slp, K1)
    stlp_ref[...] = jnp.concatenate(
        [vals, jnp.zeros((B, 128 - K1), jnp.float32)], 1)
    sti_ref[...] = jnp.concatenate(
        [idxs, jnp.zeros((B, 128 - K1), jnp.int32)], 1)
    fr = [idxs[b:b + 1, :] * B + b for b in range(B)]
    cls_r = lax.broadcasted_iota(jnp.int32, (1, B), 1)
    flat_ref[...] = jnp.concatenate(
        fr + [cls_r, jnp.zeros((1, 128 - BK - B), jnp.int32)], 1)


def _start_select(slm, pm):
    return pl.pallas_call(
        _start_select_body,
        in_specs=[
            pl.BlockSpec((B, L), lambda: (0, 0)),
            pl.BlockSpec((B, L), lambda: (0, 0)),
        ],
        out_specs=[
            pl.BlockSpec((B, 128), lambda: (0, 0)),
            pl.BlockSpec((B, 128), lambda: (0, 0)),
            pl.BlockSpec((B, L), lambda: (0, 0)),
            pl.BlockSpec((1, 128), lambda: (0, 0)),
        ],
        out_shape=[
            jax.ShapeDtypeStruct((B, 128), jnp.float32),
            jax.ShapeDtypeStruct((B, 128), jnp.int32),
            jax.ShapeDtypeStruct((B, L), jnp.float32),
            jax.ShapeDtypeStruct((1, 128), jnp.int32),
        ],
    )(slm, pm)


def _sc_gather(seq2d, flat_idx):
    mesh = plsc.VectorSubcoreMesh(core_axis_name="c", subcore_axis_name="s")

    @functools.partial(
        pl.kernel,
        mesh=mesh,
        out_type=jax.ShapeDtypeStruct((16, H), jnp.float32),
        scratch_types=[
            pltpu.VMEM((16,), jnp.int32),
            pltpu.VMEM((16, H), jnp.float32),
            pltpu.SemaphoreType.DMA,
        ],
    )
    def gat(seq_hbm, idx_hbm, out_hbm, idx_v, rows_v, sem):
        cid = lax.axis_index("c")
        sid = lax.axis_index("s")

        @pl.when((cid == 0) & (sid == 0))
        def _():
            pltpu.sync_copy(idx_hbm.at[pl.ds(0, 16)], idx_v)
            pltpu.async_copy(seq_hbm.at[idx_v], rows_v, sem).wait()
            pltpu.sync_copy(rows_v, out_hbm)

    return gat(seq2d, flat_idx)


def _end_body(seq_ref, we0a_ref, we0b_ref, rows3_ref, rows16_ref, gamma_ref,
              beta_ref, we1_ref, be1_ref, be0_ref, sp_ref, pm3_ref, wa0_ref,
              ba0_ref, wa1_ref, etlp_ref, eti_ref, cls_ref, el_s, sf_s):
    bi = pl.program_id(0)
    li = pl.program_id(1)
    x2 = seq_ref[...].reshape(LT, H)
    a = jnp.dot(x2, we0a_ref[...],
                preferred_element_type=jnp.float32)
    cb = jnp.dot(rows3_ref[...].reshape(K1, H), we0b_ref[...],
                 preferred_element_type=jnp.float32) + be0_ref[...]
    g = gamma_ref[...]
    bt = beta_ref[...]
    w1 = we1_ref[...]
    be1 = be1_ref[0, 0]
    for k in range(K1):
        ck = cb[k:k + 1, :]
        t = jnp.tanh(a + ck)
        mu = jnp.mean(t, -1, keepdims=True)
        u = t - mu
        var = jnp.mean(u * u, -1, keepdims=True)
        ln = u * lax.rsqrt(var + EPS) * g + bt
        col = jnp.dot(ln.astype(jnp.bfloat16), w1,
                      preferred_element_type=jnp.float32) + be1
        el_s[pl.ds(li * LT, LT), k:k + 1] = col

    contrib = jnp.dot(sp_ref[...].reshape(1, LT).astype(jnp.bfloat16), x2,
                      preferred_element_type=jnp.float32)

    @pl.when(li == 0)
    def _():
        sf_s[pl.ds(bi, 1), :] = jnp.zeros((1, H), jnp.float32)

    sf_s[pl.ds(bi, 1), :] += contrib

    @pl.when(li == GRID - 1)
    def _():
        el = el_s[...]
        pmb = pm3_ref[...].reshape(L, 1).astype(jnp.float32)
        elm = el * (1.0 - pmb) + NEG * pmb
        m = jnp.max(elm, axis=0, keepdims=True)
        se = jnp.sum(jnp.exp(elm - m), axis=0, keepdims=True)
        elp = elm - m - jnp.log(se)
        iota = lax.broadcasted_iota(jnp.int32, (L, K1), 0)
        cur = elp
        vals, idxs = [], []
        for _ in range(K2):
            v = jnp.max(cur, axis=0, keepdims=True)
            i = jnp.min(jnp.where(cur == v, iota, L), axis=0, keepdims=True)
            vals.append(v)
            idxs.append(i)
            cur = jnp.where(iota == i, -1e38, cur)
        valc = jnp.concatenate(vals, 0)
        idxc = jnp.concatenate(idxs, 0)
        etlp_ref[...] = jnp.concatenate(
            [jnp.concatenate([valc, jnp.zeros((8 - K2, K1), jnp.float32)], 0),
             jnp.zeros((8, 128 - K1), jnp.float32)], 1).reshape(1, 8, 128)
        eti_ref[...] = jnp.concatenate(
            [jnp.concatenate([idxc, jnp.zeros((8 - K2, K1), jnp.int32)], 0),
             jnp.zeros((8, 128 - K1), jnp.int32)], 1).reshape(1, 8, 128)

    @pl.when((bi == B - 1) & (li == GRID - 1))
    def _():
        af = jnp.concatenate(
            [sf_s[0:B, :], rows16_ref[BK:BK + B, :]], 1)
        h1 = jnp.tanh(jnp.dot(af.astype(jnp.bfloat16), wa0_ref[...],
                              preferred_element_type=jnp.float32)
                      + ba0_ref[...])
        cl = jnp.dot(h1.astype(jnp.bfloat16), wa1_ref[...],
                     preferred_element_type=jnp.float32)
        cls_ref[...] = jnp.concatenate(
            [cl, jnp.zeros((B, 127), jnp.float32)], 1)


def _end_stage(seq4, we0a, we0b, rows3, rows16, gamma, beta, we1, be1, be0,
               sp3, pm3, wa0, ba0, wa1):
    return pl.pallas_call(
        _end_body,
        grid=(B, GRID),
        in_specs=[
            pl.BlockSpec((LT, 1, 1, H), lambda b, l: (l, b, 0, 0)),
            pl.BlockSpec((H, H), lambda b, l: (0, 0)),
            pl.BlockSpec((H, H), lambda b, l: (0, 0)),
            pl.BlockSpec((1, K1, H), lambda b, l: (b, 0, 0)),
            pl.BlockSpec((16, H), lambda b, l: (0, 0)),
            pl.BlockSpec((1, H), lambda b, l: (0, 0)),
            pl.BlockSpec((1, H), lambda b, l: (0, 0)),
            pl.BlockSpec((H, 1), lambda b, l: (0, 0)),
            pl.BlockSpec((1, 1), lambda b, l: (0, 0)),
            pl.BlockSpec((1, H), lambda b, l: (0, 0)),
            pl.BlockSpec((1, 1, LT), lambda b, l: (b, 0, l)),
            pl.BlockSpec((1, L, 1), lambda b, l: (b, 0, 0)),
            pl.BlockSpec((2 * H, H), lambda b, l: (0, 0)),
            pl.BlockSpec((1, H), lambda b, l: (0, 0)),
            pl.BlockSpec((H, 1), lambda b, l: (0, 0)),
        ],
        out_specs=[
            pl.BlockSpec((1, 8, 128), lambda b, l: (b, 0, 0)),
            pl.BlockSpec((1, 8, 128), lambda b, l: (b, 0, 0)),
            pl.BlockSpec((B, 128), lambda b, l: (0, 0)),
        ],
        out_shape=[
            jax.ShapeDtypeStruct((B, 8, 128), jnp.float32),
            jax.ShapeDtypeStruct((B, 8, 128), jnp.int32),
            jax.ShapeDtypeStruct((B, 128), jnp.float32),
        ],
        scratch_shapes=[
            pltpu.VMEM((L, K1), jnp.float32),
            pltpu.VMEM((8, H), jnp.float32),
        ],
        compiler_params=pltpu.CompilerParams(
            dimension_semantics=("arbitrary", "arbitrary")),
    )(seq4, we0a, we0b, rows3, rows16, gamma, beta, we1, be1, be0, sp3, pm3,
      wa0, ba0, wa1)


def kernel(sequence_output, p_mask, Ws, bs, We0, be0, gamma, beta, We1, be1,
           Wa0, ba0, Wa1, start_n_top, end_n_top):
    bf = jnp.bfloat16
    seq_bf = sequence_output.astype(bf)
    sl = _start_logits(seq_bf, Ws.astype(bf), bs.reshape(1, 1))
    stlp_p, sti_p, start_p, flatp = _start_select(sl.T, p_mask)
    stlp = stlp_p[:, :K1]
    sti = sti_p[:, :K1]

    rows16 = _sc_gather(sequence_output.reshape(L * B, H),
                        flatp.reshape(128))
    rows3 = rows16[:BK].reshape(B, K1, H).astype(bf)

    etlp_p, eti_p, cls_p = _end_stage(
        seq_bf.reshape(L, B, 1, H), We0[:H].astype(bf), We0[H:].astype(bf),
        rows3, rows16, gamma.reshape(1, H), beta.reshape(1, H), We1.astype(bf),
        be1.reshape(1, 1), be0.reshape(1, H), start_p.reshape(B, 1, L),
        p_mask.reshape(B, L, 1), Wa0.astype(bf), ba0.reshape(1, H),
        Wa1.astype(bf))
    etlp = etlp_p[:, :K2, :K1].transpose(0, 2, 1).reshape(B, K1 * K2)
    eti = eti_p[:, :K2, :K1].transpose(0, 2, 1).reshape(B, K1 * K2)
    cls_logits = cls_p[:, 0]
    return (stlp, sti, etlp, eti, cls_logits)


# speedup vs baseline: 2.8035x; 1.1130x over previous
import functools

import jax
import jax.numpy as jnp
from jax import lax
from jax.experimental import pallas as pl
from jax.experimental.pallas import tpu as pltpu
from jax.experimental.pallas import tpu_sc as plsc

L, B, H = 2048, 2, 1024
K1, K2 = 5, 5
BK = B * K1
EPS = 1e-12
NEG = -1e30
LT = 1024
GRID = L // LT


def _topk_rows(x, k):
    r = x.shape[0]
    iota = lax.broadcasted_iota(jnp.int32, (r, L), 1)
    cur = x
    vals, idxs = [], []
    for _ in range(k):
        v = jnp.max(cur, axis=-1, keepdims=True)
        i = jnp.min(jnp.where(cur == v, iota, L), axis=-1, keepdims=True)
        vals.append(v)
        idxs.append(i)
        cur = jnp.where(iota == i, -1e38, cur)
    return jnp.concatenate(vals, 1), jnp.concatenate(idxs, 1)


def _start_select_body(seq_ref, ws_ref, bs_ref, pm_ref, seqbf_ref, stlp_ref,
                       sti_ref, sp_ref, flat_ref, sl_s):
    i = pl.program_id(0)
    x = seq_ref[...]
    xbf = x.astype(jnp.bfloat16)
    seqbf_ref[...] = xbf
    slb = jnp.dot(xbf.reshape(LT * B, H), ws_ref[...],
                  preferred_element_type=jnp.float32)
    sl_s[pl.ds(i * LT, LT), :] = slb.reshape(LT, B) + bs_ref[0, 0]

    @pl.when(i == GRID - 1)
    def _():
        sl = sl_s[...].T
        pm = pm_ref[...].astype(jnp.float32)
        slm = sl * (1.0 - pm) + NEG * pm
        m = jnp.max(slm, axis=-1, keepdims=True)
        ex = jnp.exp(slm - m)
        se = jnp.sum(ex, axis=-1, keepdims=True)
        sp_ref[...] = ex / se
        slp = slm - m - jnp.log(se)
        vals, idxs = _topk_rows(slp, K1)
        stlp_ref[...] = jnp.concatenate(
            [vals, jnp.zeros((B, 128 - K1), jnp.float32)], 1)
        sti_ref[...] = jnp.concatenate(
            [idxs, jnp.zeros((B, 128 - K1), jnp.int32)], 1)
        fr = [idxs[b:b + 1, :] * B + b for b in range(B)]
        cls_r = lax.broadcasted_iota(jnp.int32, (1, B), 1)
        flat_ref[...] = jnp.concatenate(
            fr + [cls_r, jnp.zeros((1, 128 - BK - B), jnp.int32)], 1)


def _start_select(seq, ws, bs, pm):
    return pl.pallas_call(
        _start_select_body,
        grid=(GRID,),
        in_specs=[
            pl.BlockSpec((LT, B, H), lambda i: (i, 0, 0)),
            pl.BlockSpec((H, 1), lambda i: (0, 0)),
            pl.BlockSpec((1, 1), lambda i: (0, 0)),
            pl.BlockSpec((B, L), lambda i: (0, 0)),
        ],
        out_specs=[
            pl.BlockSpec((LT, B, H), lambda i: (i, 0, 0)),
            pl.BlockSpec((B, 128), lambda i: (0, 0)),
            pl.BlockSpec((B, 128), lambda i: (0, 0)),
            pl.BlockSpec((B, L), lambda i: (0, 0)),
            pl.BlockSpec((1, 128), lambda i: (0, 0)),
        ],
        out_shape=[
            jax.ShapeDtypeStruct((L, B, H), jnp.bfloat16),
            jax.ShapeDtypeStruct((B, 128), jnp.float32),
            jax.ShapeDtypeStruct((B, 128), jnp.int32),
            jax.ShapeDtypeStruct((B, L), jnp.float32),
            jax.ShapeDtypeStruct((1, 128), jnp.int32),
        ],
        scratch_shapes=[pltpu.VMEM((L, B), jnp.float32)],
        compiler_params=pltpu.CompilerParams(
            dimension_semantics=("arbitrary",)),
    )(seq, ws, bs, pm)


def _sc_gather(seq2d, flat_idx):
    mesh = plsc.VectorSubcoreMesh(core_axis_name="c", subcore_axis_name="s")

    @functools.partial(
        pl.kernel,
        mesh=mesh,
        out_type=jax.ShapeDtypeStruct((16, H), jnp.float32),
        scratch_types=[
            pltpu.VMEM((16,), jnp.int32),
            pltpu.VMEM((16, H), jnp.float32),
            pltpu.SemaphoreType.DMA,
        ],
    )
    def gat(seq_hbm, idx_hbm, out_hbm, idx_v, rows_v, sem):
        cid = lax.axis_index("c")
        sid = lax.axis_index("s")

        @pl.when((cid == 0) & (sid == 0))
        def _():
            pltpu.sync_copy(idx_hbm.at[pl.ds(0, 16)], idx_v)
            pltpu.async_copy(seq_hbm.at[idx_v], rows_v, sem).wait()
            pltpu.sync_copy(rows_v, out_hbm)

    return gat(seq2d, flat_idx)


def _end_body(seq_ref, we0a_ref, we0b_ref, rows3_ref, rows16_ref, gamma_ref,
              beta_ref, we1_ref, be1_ref, be0_ref, sp_ref, pm3_ref, wa0_ref,
              ba0_ref, wa1_ref, etlp_ref, eti_ref, cls_ref, el_s, sf_s):
    bi = pl.program_id(0)
    li = pl.program_id(1)
    x2 = seq_ref[...].reshape(LT, H)
    a = jnp.dot(x2, we0a_ref[...],
                preferred_element_type=jnp.float32)
    cb = jnp.dot(rows3_ref[...].reshape(K1, H), we0b_ref[...],
                 preferred_element_type=jnp.float32) + be0_ref[...]
    g = gamma_ref[...]
    bt = beta_ref[...]
    w1 = we1_ref[...]
    be1 = be1_ref[0, 0]
    for k in range(K1):
        ck = cb[k:k + 1, :]
        t = jnp.tanh(a + ck)
        mu = jnp.mean(t, -1, keepdims=True)
        u = t - mu
        var = jnp.mean(u * u, -1, keepdims=True)
        ln = u * lax.rsqrt(var + EPS) * g + bt
        col = jnp.dot(ln.astype(jnp.bfloat16), w1,
                      preferred_element_type=jnp.float32) + be1
        el_s[pl.ds(li * LT, LT), k:k + 1] = col

    contrib = jnp.dot(sp_ref[...].reshape(1, LT).astype(jnp.bfloat16), x2,
                      preferred_element_type=jnp.float32)

    @pl.when(li == 0)
    def _():
        sf_s[pl.ds(bi, 1), :] = jnp.zeros((1, H), jnp.float32)

    sf_s[pl.ds(bi, 1), :] += contrib

    @pl.when(li == GRID - 1)
    def _():
        el = el_s[...]
        pmb = pm3_ref[...].reshape(L, 1).astype(jnp.float32)
        elm = el * (1.0 - pmb) + NEG * pmb
        m = jnp.max(elm, axis=0, keepdims=True)
        se = jnp.sum(jnp.exp(elm - m), axis=0, keepdims=True)
        elp = elm - m - jnp.log(se)
        iota = lax.broadcasted_iota(jnp.int32, (L, K1), 0)
        cur = elp
        vals, idxs = [], []
        for _ in range(K2):
            v = jnp.max(cur, axis=0, keepdims=True)
            i = jnp.min(jnp.where(cur == v, iota, L), axis=0, keepdims=True)
            vals.append(v)
            idxs.append(i)
            cur = jnp.where(iota == i, -1e38, cur)
        valc = jnp.concatenate(vals, 0)
        idxc = jnp.concatenate(idxs, 0)
        etlp_ref[...] = jnp.concatenate(
            [jnp.concatenate([valc, jnp.zeros((8 - K2, K1), jnp.float32)], 0),
             jnp.zeros((8, 128 - K1), jnp.float32)], 1).reshape(1, 8, 128)
        eti_ref[...] = jnp.concatenate(
            [jnp.concatenate([idxc, jnp.zeros((8 - K2, K1), jnp.int32)], 0),
             jnp.zeros((8, 128 - K1), jnp.int32)], 1).reshape(1, 8, 128)

    @pl.when((bi == B - 1) & (li == GRID - 1))
    def _():
        af = jnp.concatenate(
            [sf_s[0:B, :], rows16_ref[BK:BK + B, :]], 1)
        h1 = jnp.tanh(jnp.dot(af.astype(jnp.bfloat16), wa0_ref[...],
                              preferred_element_type=jnp.float32)
                      + ba0_ref[...])
        cl = jnp.dot(h1.astype(jnp.bfloat16), wa1_ref[...],
                     preferred_element_type=jnp.float32)
        cls_ref[...] = jnp.concatenate(
            [cl, jnp.zeros((B, 127), jnp.float32)], 1)


def _end_stage(seq4, we0a, we0b, rows3, rows16, gamma, beta, we1, be1, be0,
               sp3, pm3, wa0, ba0, wa1):
    return pl.pallas_call(
        _end_body,
        grid=(B, GRID),
        in_specs=[
            pl.BlockSpec((LT, 1, 1, H), lambda b, l: (l, b, 0, 0)),
            pl.BlockSpec((H, H), lambda b, l: (0, 0)),
            pl.BlockSpec((H, H), lambda b, l: (0, 0)),
            pl.BlockSpec((1, K1, H), lambda b, l: (b, 0, 0)),
            pl.BlockSpec((16, H), lambda b, l: (0, 0)),
            pl.BlockSpec((1, H), lambda b, l: (0, 0)),
            pl.BlockSpec((1, H), lambda b, l: (0, 0)),
            pl.BlockSpec((H, 1), lambda b, l: (0, 0)),
            pl.BlockSpec((1, 1), lambda b, l: (0, 0)),
            pl.BlockSpec((1, H), lambda b, l: (0, 0)),
            pl.BlockSpec((1, 1, LT), lambda b, l: (b, 0, l)),
            pl.BlockSpec((1, L, 1), lambda b, l: (b, 0, 0)),
            pl.BlockSpec((2 * H, H), lambda b, l: (0, 0)),
            pl.BlockSpec((1, H), lambda b, l: (0, 0)),
            pl.BlockSpec((H, 1), lambda b, l: (0, 0)),
        ],
        out_specs=[
            pl.BlockSpec((1, 8, 128), lambda b, l: (b, 0, 0)),
            pl.BlockSpec((1, 8, 128), lambda b, l: (b, 0, 0)),
            pl.BlockSpec((B, 128), lambda b, l: (0, 0)),
        ],
        out_shape=[
            jax.ShapeDtypeStruct((B, 8, 128), jnp.float32),
            jax.ShapeDtypeStruct((B, 8, 128), jnp.int32),
            jax.ShapeDtypeStruct((B, 128), jnp.float32),
        ],
        scratch_shapes=[
            pltpu.VMEM((L, K1), jnp.float32),
            pltpu.VMEM((8, H), jnp.float32),
        ],
        compiler_params=pltpu.CompilerParams(
            dimension_semantics=("arbitrary", "arbitrary")),
    )(seq4, we0a, we0b, rows3, rows16, gamma, beta, we1, be1, be0, sp3, pm3,
      wa0, ba0, wa1)


def kernel(sequence_output, p_mask, Ws, bs, We0, be0, gamma, beta, We1, be1,
           Wa0, ba0, Wa1, start_n_top, end_n_top):
    bf = jnp.bfloat16
    seq_bf, stlp_p, sti_p, start_p, flatp = _start_select(
        sequence_output, Ws.astype(bf), bs.reshape(1, 1), p_mask)
    stlp = stlp_p[:, :K1]
    sti = sti_p[:, :K1]

    rows16 = _sc_gather(sequence_output.reshape(L * B, H),
                        flatp.reshape(128))
    rows3 = rows16[:BK].reshape(B, K1, H).astype(bf)

    etlp_p, eti_p, cls_p = _end_stage(
        seq_bf.reshape(L, B, 1, H), We0[:H].astype(bf), We0[H:].astype(bf),
        rows3, rows16, gamma.reshape(1, H), beta.reshape(1, H), We1.astype(bf),
        be1.reshape(1, 1), be0.reshape(1, H), start_p.reshape(B, 1, L),
        p_mask.reshape(B, L, 1), Wa0.astype(bf), ba0.reshape(1, H),
        Wa1.astype(bf))
    etlp = etlp_p[:, :K2, :K1].transpose(0, 2, 1).reshape(B, K1 * K2)
    eti = eti_p[:, :K2, :K1].transpose(0, 2, 1).reshape(B, K1 * K2)
    cls_logits = cls_p[:, 0]
    return (stlp, sti, etlp, eti, cls_logits)


# speedup vs baseline: 2.8883x; 1.0302x over previous
import functools

import jax
import jax.numpy as jnp
from jax import lax
from jax.experimental import pallas as pl
from jax.experimental.pallas import tpu as pltpu
from jax.experimental.pallas import tpu_sc as plsc

L, B, H = 2048, 2, 1024
K1, K2 = 5, 5
BK = B * K1
EPS = 1e-12
NEG = -1e30
LT = 1024
GRID = L // LT


def _topk_rows(x, k):
    r = x.shape[0]
    iota = lax.broadcasted_iota(jnp.int32, (r, L), 1)
    cur = x
    vals, idxs = [], []
    for _ in range(k):
        v = jnp.max(cur, axis=-1, keepdims=True)
        i = jnp.min(jnp.where(cur == v, iota, L), axis=-1, keepdims=True)
        vals.append(v)
        idxs.append(i)
        cur = jnp.where(iota == i, -1e38, cur)
    return jnp.concatenate(vals, 1), jnp.concatenate(idxs, 1)


def _start_select_body(seq_ref, ws_ref, bs_ref, pm_ref, seqbf_ref, stlp_ref,
                       sti_ref, sp_ref, flat_ref, sl_s):
    i = pl.program_id(0)
    x = seq_ref[...]
    xbf = x.astype(jnp.bfloat16)
    seqbf_ref[...] = xbf
    slb = jnp.dot(xbf.reshape(LT * B, H), ws_ref[...].astype(jnp.bfloat16),
                  preferred_element_type=jnp.float32)
    sl_s[pl.ds(i * LT, LT), :] = slb.reshape(LT, B) + bs_ref[0, 0]

    @pl.when(i == GRID - 1)
    def _():
        sl = sl_s[...].T
        pm = pm_ref[...].astype(jnp.float32)
        slm = sl * (1.0 - pm) + NEG * pm
        m = jnp.max(slm, axis=-1, keepdims=True)
        ex = jnp.exp(slm - m)
        se = jnp.sum(ex, axis=-1, keepdims=True)
        sp_ref[...] = ex / se
        slp = slm - m - jnp.log(se)
        vals, idxs = _topk_rows(slp, K1)
        stlp_ref[...] = jnp.concatenate(
            [vals, jnp.zeros((B, 128 - K1), jnp.float32)], 1)
        sti_ref[...] = jnp.concatenate(
            [idxs, jnp.zeros((B, 128 - K1), jnp.int32)], 1)
        fr = [idxs[b:b + 1, :] * B + b for b in range(B)]
        cls_r = lax.broadcasted_iota(jnp.int32, (1, B), 1)
        flat_ref[...] = jnp.concatenate(
            fr + [cls_r, jnp.zeros((1, 128 - BK - B), jnp.int32)], 1)


def _start_select(seq, ws, bs, pm):
    return pl.pallas_call(
        _start_select_body,
        grid=(GRID,),
        in_specs=[
            pl.BlockSpec((LT, B, H), lambda i: (i, 0, 0)),
            pl.BlockSpec((H, 1), lambda i: (0, 0)),
            pl.BlockSpec((1, 1), lambda i: (0, 0)),
            pl.BlockSpec((B, L), lambda i: (0, 0)),
        ],
        out_specs=[
            pl.BlockSpec((LT, B, H), lambda i: (i, 0, 0)),
            pl.BlockSpec((B, 128), lambda i: (0, 0)),
            pl.BlockSpec((B, 128), lambda i: (0, 0)),
            pl.BlockSpec((B, L), lambda i: (0, 0)),
            pl.BlockSpec((1, 128), lambda i: (0, 0)),
        ],
        out_shape=[
            jax.ShapeDtypeStruct((L, B, H), jnp.bfloat16),
            jax.ShapeDtypeStruct((B, 128), jnp.float32),
            jax.ShapeDtypeStruct((B, 128), jnp.int32),
            jax.ShapeDtypeStruct((B, L), jnp.float32),
            jax.ShapeDtypeStruct((1, 128), jnp.int32),
        ],
        scratch_shapes=[pltpu.VMEM((L, B), jnp.float32)],
        compiler_params=pltpu.CompilerParams(
            dimension_semantics=("arbitrary",)),
    )(seq, ws, bs, pm)


def _sc_gather(seq2d, flat_idx):
    mesh = plsc.VectorSubcoreMesh(core_axis_name="c", subcore_axis_name="s")

    @functools.partial(
        pl.kernel,
        mesh=mesh,
        out_type=jax.ShapeDtypeStruct((16, H), jnp.float32),
        scratch_types=[
            pltpu.VMEM((16,), jnp.int32),
            pltpu.VMEM((16, H), jnp.float32),
            pltpu.SemaphoreType.DMA,
        ],
    )
    def gat(seq_hbm, idx_hbm, out_hbm, idx_v, rows_v, sem):
        cid = lax.axis_index("c")
        sid = lax.axis_index("s")

        @pl.when((cid == 0) & (sid == 0))
        def _():
            pltpu.sync_copy(idx_hbm.at[pl.ds(0, 16)], idx_v)
            pltpu.async_copy(seq_hbm.at[idx_v], rows_v, sem).wait()
            pltpu.sync_copy(rows_v, out_hbm)

    return gat(seq2d, flat_idx)


def _end_body(seq_ref, we0_ref, rows3_ref, rows16_ref, gamma_ref,
              beta_ref, we1_ref, be1_ref, be0_ref, sp_ref, pm3_ref, wa0_ref,
              ba0_ref, wa1_ref, etlp_ref, eti_ref, cls_ref, el_s, sf_s,
              w0a_s, w0b_s):
    bi = pl.program_id(0)
    li = pl.program_id(1)

    @pl.when((bi == 0) & (li == 0))
    def _():
        w0a_s[...] = we0_ref[0:H, :].astype(jnp.bfloat16)
        w0b_s[...] = we0_ref[H:2 * H, :].astype(jnp.bfloat16)

    x2 = seq_ref[...].reshape(LT, H)
    a = jnp.dot(x2, w0a_s[...],
                preferred_element_type=jnp.float32)
    rowsb = rows3_ref[...].reshape(K1, H)
    cb = jnp.dot(rowsb.astype(jnp.bfloat16), w0b_s[...],
                 preferred_element_type=jnp.float32) + be0_ref[...]
    g = gamma_ref[...]
    bt = beta_ref[...]
    w1 = we1_ref[...].astype(jnp.bfloat16)
    be1 = be1_ref[0, 0]
    for k in range(K1):
        ck = cb[k:k + 1, :]
        t = jnp.tanh(a + ck)
        mu = jnp.mean(t, -1, keepdims=True)
        u = t - mu
        var = jnp.mean(u * u, -1, keepdims=True)
        ln = u * lax.rsqrt(var + EPS) * g + bt
        col = jnp.dot(ln.astype(jnp.bfloat16), w1,
                      preferred_element_type=jnp.float32) + be1
        el_s[pl.ds(li * LT, LT), k:k + 1] = col

    contrib = jnp.dot(sp_ref[...].reshape(1, LT).astype(jnp.bfloat16), x2,
                      preferred_element_type=jnp.float32)

    @pl.when(li == 0)
    def _():
        sf_s[pl.ds(bi, 1), :] = jnp.zeros((1, H), jnp.float32)

    sf_s[pl.ds(bi, 1), :] += contrib

    @pl.when(li == GRID - 1)
    def _():
        el = el_s[...]
        pmb = pm3_ref[...].reshape(L, 1).astype(jnp.float32)
        elm = el * (1.0 - pmb) + NEG * pmb
        m = jnp.max(elm, axis=0, keepdims=True)
        se = jnp.sum(jnp.exp(elm - m), axis=0, keepdims=True)
        elp = elm - m - jnp.log(se)
        iota = lax.broadcasted_iota(jnp.int32, (L, K1), 0)
        cur = elp
        vals, idxs = [], []
        for _ in range(K2):
            v = jnp.max(cur, axis=0, keepdims=True)
            i = jnp.min(jnp.where(cur == v, iota, L), axis=0, keepdims=True)
            vals.append(v)
            idxs.append(i)
            cur = jnp.where(iota == i, -1e38, cur)
        valc = jnp.concatenate(vals, 0)
        idxc = jnp.concatenate(idxs, 0)
        etlp_ref[...] = jnp.concatenate(
            [jnp.concatenate([valc, jnp.zeros((8 - K2, K1), jnp.float32)], 0),
             jnp.zeros((8, 128 - K1), jnp.float32)], 1).reshape(1, 8, 128)
        eti_ref[...] = jnp.concatenate(
            [jnp.concatenate([idxc, jnp.zeros((8 - K2, K1), jnp.int32)], 0),
             jnp.zeros((8, 128 - K1), jnp.int32)], 1).reshape(1, 8, 128)

    @pl.when((bi == B - 1) & (li == GRID - 1))
    def _():
        af = jnp.concatenate(
            [sf_s[0:B, :], rows16_ref[BK:BK + B, :]], 1)
        h1 = jnp.tanh(jnp.dot(af.astype(jnp.bfloat16),
                              wa0_ref[...].astype(jnp.bfloat16),
                              preferred_element_type=jnp.float32)
                      + ba0_ref[...])
        cl = jnp.dot(h1.astype(jnp.bfloat16),
                     wa1_ref[...].astype(jnp.bfloat16),
                     preferred_element_type=jnp.float32)
        cls_ref[...] = jnp.concatenate(
            [cl, jnp.zeros((B, 127), jnp.float32)], 1)


def _end_stage(seq4, we0, rows3, rows16, gamma, beta, we1, be1, be0,
               sp3, pm3, wa0, ba0, wa1):
    return pl.pallas_call(
        _end_body,
        grid=(B, GRID),
        in_specs=[
            pl.BlockSpec((LT, 1, 1, H), lambda b, l: (l, b, 0, 0)),
            pl.BlockSpec((2 * H, H), lambda b, l: (0, 0)),
            pl.BlockSpec((1, K1, H), lambda b, l: (b, 0, 0)),
            pl.BlockSpec((16, H), lambda b, l: (0, 0)),
            pl.BlockSpec((1, H), lambda b, l: (0, 0)),
            pl.BlockSpec((1, H), lambda b, l: (0, 0)),
            pl.BlockSpec((H, 1), lambda b, l: (0, 0)),
            pl.BlockSpec((1, 1), lambda b, l: (0, 0)),
            pl.BlockSpec((1, H), lambda b, l: (0, 0)),
            pl.BlockSpec((1, 1, LT), lambda b, l: (b, 0, l)),
            pl.BlockSpec((1, L, 1), lambda b, l: (b, 0, 0)),
            pl.BlockSpec((2 * H, H), lambda b, l: (0, 0)),
            pl.BlockSpec((1, H), lambda b, l: (0, 0)),
            pl.BlockSpec((H, 1), lambda b, l: (0, 0)),
        ],
        out_specs=[
            pl.BlockSpec((1, 8, 128), lambda b, l: (b, 0, 0)),
            pl.BlockSpec((1, 8, 128), lambda b, l: (b, 0, 0)),
            pl.BlockSpec((B, 128), lambda b, l: (0, 0)),
        ],
        out_shape=[
            jax.ShapeDtypeStruct((B, 8, 128), jnp.float32),
            jax.ShapeDtypeStruct((B, 8, 128), jnp.int32),
            jax.ShapeDtypeStruct((B, 128), jnp.float32),
        ],
        scratch_shapes=[
            pltpu.VMEM((L, K1), jnp.float32),
            pltpu.VMEM((8, H), jnp.float32),
            pltpu.VMEM((H, H), jnp.bfloat16),
            pltpu.VMEM((H, H), jnp.bfloat16),
        ],
        compiler_params=pltpu.CompilerParams(
            dimension_semantics=("arbitrary", "arbitrary")),
    )(seq4, we0, rows3, rows16, gamma, beta, we1, be1, be0, sp3, pm3,
      wa0, ba0, wa1)


def kernel(sequence_output, p_mask, Ws, bs, We0, be0, gamma, beta, We1, be1,
           Wa0, ba0, Wa1, start_n_top, end_n_top):
    bf = jnp.bfloat16
    seq_bf, stlp_p, sti_p, start_p, flatp = _start_select(
        sequence_output, Ws, bs.reshape(1, 1), p_mask)
    stlp = stlp_p[:, :K1]
    sti = sti_p[:, :K1]

    rows16 = _sc_gather(sequence_output.reshape(L * B, H),
                        flatp.reshape(128))

    etlp_p, eti_p, cls_p = _end_stage(
        seq_bf.reshape(L, B, 1, H), We0, rows16[:BK].reshape(B, K1, H),
        rows16, gamma.reshape(1, H), beta.reshape(1, H), We1,
        be1.reshape(1, 1), be0.reshape(1, H), start_p.reshape(B, 1, L),
        p_mask.reshape(B, L, 1), Wa0, ba0.reshape(1, H), Wa1)
    etlp = etlp_p[:, :K2, :K1].transpose(0, 2, 1).reshape(B, K1 * K2)
    eti = eti_p[:, :K2, :K1].transpose(0, 2, 1).reshape(B, K1 * K2)
    cls_logits = cls_p[:, 0]
    return (stlp, sti, etlp, eti, cls_logits)
